# baseline (speedup 1.0000x reference)
import functools

import jax
import jax.numpy as jnp
from jax import lax
from jax.experimental import pallas as pl
from jax.experimental.pallas import tpu as pltpu
from jax.experimental.pallas import tpu_sc as plsc


def _dist_body(x_ref, et_ref, xn_ref, idx_ref, xnb_s, x2_s, enb_s, e2_s,
               mv_s, mi_s, *, T, D, BM, M):
    n = pl.program_id(0)
    m = pl.program_id(1)
    nm = pl.num_programs(1)

    @pl.when(m == 0)
    def _init():
        xb = x_ref[0]
        mu = jnp.mean(xb, axis=0, keepdims=True)
        std = jnp.std(xb, axis=0, keepdims=True, ddof=1)
        xn = (xb - mu) / (std + 1e-5)
        xn_ref[0] = xn
        xnb_s[...] = xn.astype(jnp.bfloat16)
        x2_s[...] = jnp.sum(xn * xn, axis=1, keepdims=True)
        mv_s[...] = jnp.full((T, 1), jnp.inf, dtype=jnp.float32)
        mi_s[...] = jnp.zeros((T, 1), dtype=jnp.int32)

    @pl.when(n == 0)
    def _embnorm():
        et = et_ref[:, pl.ds(m * BM, BM)]
        nrm = jnp.sqrt(jnp.sum(et * et, axis=0, keepdims=True))
        en = et / (nrm + 1e-4)
        e2_s[:, pl.ds(m * BM, BM)] = jnp.sum(en * en, axis=0, keepdims=True)
        enb_s[:, pl.ds(m * BM, BM)] = en.astype(jnp.bfloat16)

    s = lax.dot_general(xnb_s[...], enb_s[:, pl.ds(m * BM, BM)],
                        (((1,), (0,)), ((), ())),
                        preferred_element_type=jnp.float32)
    dist = (e2_s[:, pl.ds(m * BM, BM)] + x2_s[...]) - 2.0 * s
    rowmin = jnp.min(dist, axis=1, keepdims=True)
    col = lax.broadcasted_iota(jnp.int32, (T, BM), 1)
    cand = jnp.where(dist == rowmin, col, M)
    barg = jnp.min(cand, axis=1, keepdims=True) + m * BM
    prev = mv_s[...]
    better = rowmin < prev
    mi_s[...] = jnp.where(better, barg, mi_s[...])
    mv_s[...] = jnp.where(better, rowmin, prev)

    @pl.when(m == nm - 1)
    def _fin():
        idx_ref[0] = mi_s[...]


def _dist_argmin(x, emb_t):
    N, T, D = x.shape
    M = emb_t.shape[1]
    BM = 1024
    grid = (N, M // BM)
    return pl.pallas_call(
        functools.partial(_dist_body, T=T, D=D, BM=BM, M=M),
        grid=grid,
        in_specs=[
            pl.BlockSpec((1, T, D), lambda n, m: (n, 0, 0)),
            pl.BlockSpec((D, M), lambda n, m: (0, 0)),
        ],
        out_specs=[
            pl.BlockSpec((1, T, D), lambda n, m: (n, 0, 0)),
            pl.BlockSpec((1, T, 1), lambda n, m: (n, 0, 0)),
        ],
        out_shape=[
            jax.ShapeDtypeStruct((N, T, D), jnp.float32),
            jax.ShapeDtypeStruct((N, T, 1), jnp.int32),
        ],
        scratch_shapes=[
            pltpu.VMEM((T, D), jnp.bfloat16),
            pltpu.VMEM((T, 1), jnp.float32),
            pltpu.VMEM((D, M), jnp.bfloat16),
            pltpu.VMEM((1, M), jnp.float32),
            pltpu.VMEM((T, 1), jnp.float32),
            pltpu.VMEM((T, 1), jnp.int32),
        ],
    )(x, emb_t)


def _sc_gather(table, idx_flat):
    M, D = table.shape
    B = idx_flat.shape[0]
    info = plsc.get_sparse_core_info()
    NC, NS = info.num_cores, info.num_subcores
    NW = NC * NS
    b_per_w = B // NW
    mesh = plsc.VectorSubcoreMesh(core_axis_name="c", subcore_axis_name="s")

    @functools.partial(
        pl.kernel, mesh=mesh,
        out_type=jax.ShapeDtypeStruct((B, D), jnp.float32),
        scratch_types=[
            pltpu.VMEM((b_per_w,), jnp.int32),
            pltpu.VMEM((b_per_w, D), jnp.float32),
            pltpu.SemaphoreType.DMA,
        ],
    )
    def gather_k(table_hbm, idx_hbm, out_hbm, idx_v, rows_v, sem):
        wid = lax.axis_index("s") * NC + lax.axis_index("c")
        base = wid * b_per_w
        pltpu.sync_copy(idx_hbm.at[pl.ds(base, b_per_w)], idx_v)
        pltpu.async_copy(table_hbm.at[idx_v], rows_v, sem).wait()
        pltpu.sync_copy(rows_v, out_hbm.at[pl.ds(base, b_per_w)])

    return gather_k(table, idx_flat)


def _stats_body(xn_ref, q_ref, idx_ref, qout_ref, loss_ref, perp_ref,
                sum_s, cnt_s, *, N, T, D, M):
    n = pl.program_id(0)

    @pl.when(n == 0)
    def _init():
        sum_s[...] = jnp.zeros((1, 1), dtype=jnp.float32)
        cnt_s[...] = jnp.zeros((1, M), dtype=jnp.float32)

    xn = xn_ref[0]
    q = q_ref[0]
    d = xn - q
    sum_s[...] += jnp.sum(d * d, axis=(0, 1), keepdims=True)
    t = xn + (q - xn)
    qout_ref[0] = (t + q) / 2.0
    idxb = idx_ref[0]
    CB = 1024
    for j in range(M // CB):
        codes = lax.broadcasted_iota(jnp.int32, (T, CB), 1) + j * CB
        hits = (idxb == codes).astype(jnp.float32)
        cnt_s[:, j * CB:(j + 1) * CB] += jnp.sum(hits, axis=0, keepdims=True)

    @pl.when(n == pl.num_programs(0) - 1)
    def _fin():
        loss_ref[...] = sum_s[...] / (N * T * D)
        p = cnt_s[...] / (N * T)
        ent = jnp.sum(p * jnp.log(p + 1e-10), axis=(0, 1), keepdims=True)
        perp_ref[...] = jnp.exp(-ent)


def _stats(xn, q, idx, M):
    N, T, D = xn.shape
    return pl.pallas_call(
        functools.partial(_stats_body, N=N, T=T, D=D, M=M),
        grid=(N,),
        in_specs=[
            pl.BlockSpec((1, T, D), lambda n: (n, 0, 0)),
            pl.BlockSpec((1, T, D), lambda n: (n, 0, 0)),
            pl.BlockSpec((1, T, 1), lambda n: (n, 0, 0)),
        ],
        out_specs=[
            pl.BlockSpec((1, T, D), lambda n: (n, 0, 0)),
            pl.BlockSpec((1, 1), lambda n: (0, 0)),
            pl.BlockSpec((1, 1), lambda n: (0, 0)),
        ],
        out_shape=[
            jax.ShapeDtypeStruct((N, T, D), jnp.float32),
            jax.ShapeDtypeStruct((1, 1), jnp.float32),
            jax.ShapeDtypeStruct((1, 1), jnp.float32),
        ],
        scratch_shapes=[
            pltpu.VMEM((1, 1), jnp.float32),
            pltpu.VMEM((1, M), jnp.float32),
        ],
    )(xn, q, idx)


def kernel(x, embedding):
    N, T, D = x.shape
    xn, idx = _dist_argmin(x, embedding.T)
    q = _sc_gather(embedding, idx.reshape(-1))
    qout, loss, perp = _stats(xn, q.reshape(N, T, D), idx, embedding.shape[0])
    return qout, loss.reshape(()), perp.reshape(())


# speedup vs baseline: 1.2425x; 1.2425x over previous
import functools

import jax
import jax.numpy as jnp
from jax import lax
from jax.experimental import pallas as pl
from jax.experimental.pallas import tpu as pltpu
from jax.experimental.pallas import tpu_sc as plsc


def _dist_body(x_ref, et_ref, xn_ref, idx_ref, xnb_s, x2_s, enb_s, e2_s,
               mv_s, mi_s, *, T, D, BM, M):
    n = pl.program_id(0)
    m = pl.program_id(1)
    nm = pl.num_programs(1)

    @pl.when(m == 0)
    def _init():
        xb = x_ref[0]
        mu = jnp.mean(xb, axis=0, keepdims=True)
        std = jnp.std(xb, axis=0, keepdims=True, ddof=1)
        xn = (xb - mu) / (std + 1e-5)
        xn_ref[0] = xn
        xnb_s[...] = xn.astype(jnp.bfloat16)
        x2_s[...] = jnp.sum(xn * xn, axis=1, keepdims=True)
        mv_s[...] = jnp.full((T, 1), jnp.inf, dtype=jnp.float32)
        mi_s[...] = jnp.zeros((T, 1), dtype=jnp.int32)

    @pl.when(n == 0)
    def _embnorm():
        et = et_ref[:, pl.ds(m * BM, BM)]
        nrm = jnp.sqrt(jnp.sum(et * et, axis=0, keepdims=True))
        en = et / (nrm + 1e-4)
        e2_s[:, pl.ds(m * BM, BM)] = jnp.sum(en * en, axis=0, keepdims=True)
        enb_s[:, pl.ds(m * BM, BM)] = en.astype(jnp.bfloat16)

    s = lax.dot_general(xnb_s[...], enb_s[:, pl.ds(m * BM, BM)],
                        (((1,), (0,)), ((), ())),
                        preferred_element_type=jnp.float32)
    dist = (e2_s[:, pl.ds(m * BM, BM)] + x2_s[...]) - 2.0 * s
    rowmin = jnp.min(dist, axis=1, keepdims=True)
    colf = lax.broadcasted_iota(jnp.int32, (1, BM), 1).astype(jnp.float32)
    candf = jnp.where(dist == rowmin, colf, jnp.float32(BM))
    barg = jnp.min(candf, axis=1, keepdims=True).astype(jnp.int32) + m * BM
    prev = mv_s[...]
    better = rowmin < prev
    mi_s[...] = jnp.where(better, barg, mi_s[...])
    mv_s[...] = jnp.where(better, rowmin, prev)

    @pl.when(m == nm - 1)
    def _fin():
        idx_ref[0] = mi_s[...]


def _dist_argmin(x, emb_t):
    N, T, D = x.shape
    M = emb_t.shape[1]
    BM = 1024
    grid = (N, M // BM)
    return pl.pallas_call(
        functools.partial(_dist_body, T=T, D=D, BM=BM, M=M),
        grid=grid,
        in_specs=[
            pl.BlockSpec((1, T, D), lambda n, m: (n, 0, 0)),
            pl.BlockSpec((D, M), lambda n, m: (0, 0)),
        ],
        out_specs=[
            pl.BlockSpec((1, T, D), lambda n, m: (n, 0, 0)),
            pl.BlockSpec((1, T, 1), lambda n, m: (n, 0, 0)),
        ],
        out_shape=[
            jax.ShapeDtypeStruct((N, T, D), jnp.float32),
            jax.ShapeDtypeStruct((N, T, 1), jnp.int32),
        ],
        scratch_shapes=[
            pltpu.VMEM((T, D), jnp.bfloat16),
            pltpu.VMEM((T, 1), jnp.float32),
            pltpu.VMEM((D, M), jnp.bfloat16),
            pltpu.VMEM((1, M), jnp.float32),
            pltpu.VMEM((T, 1), jnp.float32),
            pltpu.VMEM((T, 1), jnp.int32),
        ],
    )(x, emb_t)


def _sc_gather(table, idx_flat):
    M, D = table.shape
    B = idx_flat.shape[0]
    info = plsc.get_sparse_core_info()
    NC, NS = info.num_cores, info.num_subcores
    NW = NC * NS
    b_per_w = B // NW
    mesh = plsc.VectorSubcoreMesh(core_axis_name="c", subcore_axis_name="s")

    @functools.partial(
        pl.kernel, mesh=mesh,
        out_type=jax.ShapeDtypeStruct((B, D), jnp.float32),
        scratch_types=[
            pltpu.VMEM((b_per_w,), jnp.int32),
            pltpu.VMEM((b_per_w, D), jnp.float32),
            pltpu.SemaphoreType.DMA,
        ],
    )
    def gather_k(table_hbm, idx_hbm, out_hbm, idx_v, rows_v, sem):
        wid = lax.axis_index("s") * NC + lax.axis_index("c")
        base = wid * b_per_w
        pltpu.sync_copy(idx_hbm.at[pl.ds(base, b_per_w)], idx_v)
        pltpu.async_copy(table_hbm.at[idx_v], rows_v, sem).wait()
        pltpu.sync_copy(rows_v, out_hbm.at[pl.ds(base, b_per_w)])

    return gather_k(table, idx_flat)


def _stats_body(xn_ref, q_ref, idx_ref, qout_ref, loss_ref, perp_ref,
                sum_s, cnt_s, *, N, T, D, M):
    n = pl.program_id(0)

    MH, ML = 64, 128

    @pl.when(n == 0)
    def _init():
        sum_s[...] = jnp.zeros((1, 1), dtype=jnp.float32)
        cnt_s[...] = jnp.zeros((MH, ML), dtype=jnp.float32)

    xn = xn_ref[0]
    q = q_ref[0]
    d = xn - q
    sum_s[...] += jnp.sum(d * d, axis=(0, 1), keepdims=True)
    t = xn + (q - xn)
    qout_ref[0] = (t + q) / 2.0
    idxb = idx_ref[0]
    hi = idxb // ML
    lo = idxb - hi * ML
    ohh = (hi == lax.broadcasted_iota(jnp.int32, (T, MH), 1)).astype(jnp.float32)
    ohl = (lo == lax.broadcasted_iota(jnp.int32, (T, ML), 1)).astype(jnp.float32)
    cnt_s[...] += lax.dot_general(ohh, ohl, (((0,), (0,)), ((), ())),
                                  preferred_element_type=jnp.float32)

    @pl.when(n == pl.num_programs(0) - 1)
    def _fin():
        loss_ref[...] = sum_s[...] / (N * T * D)
        p = cnt_s[...] / (N * T)
        ent = jnp.sum(p * jnp.log(p + 1e-10), axis=(0, 1), keepdims=True)
        perp_ref[...] = jnp.exp(-ent)


def _stats(xn, q, idx, M):
    N, T, D = xn.shape
    return pl.pallas_call(
        functools.partial(_stats_body, N=N, T=T, D=D, M=M),
        grid=(N,),
        in_specs=[
            pl.BlockSpec((1, T, D), lambda n: (n, 0, 0)),
            pl.BlockSpec((1, T, D), lambda n: (n, 0, 0)),
            pl.BlockSpec((1, T, 1), lambda n: (n, 0, 0)),
        ],
        out_specs=[
            pl.BlockSpec((1, T, D), lambda n: (n, 0, 0)),
            pl.BlockSpec((1, 1), lambda n: (0, 0)),
            pl.BlockSpec((1, 1), lambda n: (0, 0)),
        ],
        out_shape=[
            jax.ShapeDtypeStruct((N, T, D), jnp.float32),
            jax.ShapeDtypeStruct((1, 1), jnp.float32),
            jax.ShapeDtypeStruct((1, 1), jnp.float32),
        ],
        scratch_shapes=[
            pltpu.VMEM((1, 1), jnp.float32),
            pltpu.VMEM((64, 128), jnp.float32),
        ],
    )(xn, q, idx)


def kernel(x, embedding):
    N, T, D = x.shape
    xn, idx = _dist_argmin(x, embedding.T)
    q = _sc_gather(embedding, idx.reshape(-1))
    qout, loss, perp = _stats(xn, q.reshape(N, T, D), idx, embedding.shape[0])
    return qout, loss.reshape(()), perp.reshape(())
